# baseline (speedup 1.0000x reference)
import functools

import jax
import jax.numpy as jnp
from jax import lax
from jax.experimental import pallas as pl
from jax.experimental.pallas import tpu as pltpu
from jax.experimental.pallas import tpu_sc as plsc

N_VALID = 10000
NP = 10240
EP = 163840
NBLK = EP // 128
BPT = 80
IRT = 40
RPS = NP // 16
BR = 512
GRID = NP // BR
NBUF = 4


def _make_sc_agg(n_chunks, with_deg):
    mesh = plsc.VectorSubcoreMesh(core_axis_name="c", subcore_axis_name="s")
    out_type = [jax.ShapeDtypeStruct((n_chunks, 2, NP, 128), jnp.float32)]
    if with_deg:
        out_type.append(jax.ShapeDtypeStruct((2, NP, 128), jnp.float32))
    scratch = [
        pltpu.VMEM((IRT, 128), jnp.int32),
        pltpu.VMEM((IRT, 128), jnp.int32),
        pltpu.VMEM((NBUF, 64, 128), jnp.float32),
        pltpu.VMEM_SHARED((NP, 128), jnp.float32),
    ] + [pltpu.SemaphoreType.DMA] * (2 * NBUF)

    @functools.partial(pl.kernel, mesh=mesh, out_type=tuple(out_type),
                       scratch_types=tuple(scratch))
    def sc_agg(*refs):
        tables = refs[:n_chunks]
        src_hbm, dst_hbm, z_hbm = (refs[n_chunks], refs[n_chunks + 1],
                                   refs[n_chunks + 2])
        k = n_chunks + 3
        out = refs[k]
        k += 1
        if with_deg:
            deg_out = refs[k]
            k += 1
        src_v, dst_v, rows = refs[k], refs[k + 1], refs[k + 2]
        acc = refs[k + 3]
        sems = refs[k + 4:k + 4 + NBUF]
        ssems = refs[k + 4 + NBUF:k + 4 + 2 * NBUF]

        scid = lax.axis_index("c")
        sid = lax.axis_index("s")
        tid = scid * 16 + sid

        pltpu.sync_copy(src_hbm.at[pl.ds(tid * IRT, IRT)], src_v)
        pltpu.sync_copy(dst_hbm.at[pl.ds(tid * IRT, IRT)], dst_v)

        def sidx(i):
            return src_v.at[i >> 1, pl.ds((i & 1) * 64, 64)]

        def didx(i):
            return dst_v.at[i >> 1, pl.ds((i & 1) * 64, 64)]

        def reset_acc():
            for r in range(RPS // 128):
                pltpu.sync_copy(z_hbm, acc.at[pl.ds(sid * RPS + r * 128, 128)])

        if with_deg:
            def orow(i, _):
                for j in range(8):
                    rows[0, i, pl.ds(j * 16, 16)] = jnp.ones((16,), jnp.float32)
                return 0
            lax.fori_loop(0, 64, orow, 0)
            reset_acc()
            plsc.subcore_barrier()

            def deg_body(i, _):
                pltpu.sync_copy(rows.at[0], acc.at[didx(i)], add=True)
                return 0
            lax.fori_loop(0, BPT, deg_body, 0)
            plsc.subcore_barrier()
            pltpu.sync_copy(acc.at[pl.ds(sid * RPS, RPS)],
                            deg_out.at[scid, pl.ds(sid * RPS, RPS)])
            plsc.subcore_barrier()

        for c in range(n_chunks):
            tab = tables[c]
            for b in range(2):
                pltpu.async_copy(tab.at[sidx(b)], rows.at[b], sems[b])
            reset_acc()
            plsc.subcore_barrier()

            def body(j4, _, _t=tab):
                for u in range(NBUF):
                    i = j4 * NBUF + u
                    b = u
                    bn = (u + 2) % NBUF
                    pltpu.make_async_copy(
                        _t.at[sidx(i)], rows.at[b], sems[b]).wait()
                    pltpu.async_copy(
                        rows.at[b], acc.at[didx(i)], ssems[b], add=True)

                    @pl.when(i + 2 < BPT)
                    def _(b=b, bn=bn, i=i):
                        @pl.when(i >= 2)
                        def _():
                            pltpu.make_async_copy(
                                rows.at[bn], acc.at[didx(i - 2)],
                                ssems[bn]).wait()
                        pltpu.async_copy(
                            _t.at[sidx(i + 2)], rows.at[bn], sems[bn])
                return 0
            lax.fori_loop(0, BPT // NBUF, body, 0)
            for i in range(BPT - NBUF, BPT):
                pltpu.make_async_copy(
                    rows.at[i % NBUF], acc.at[didx(i)],
                    ssems[i % NBUF]).wait()
            plsc.subcore_barrier()
            pltpu.sync_copy(acc.at[pl.ds(sid * RPS, RPS)],
                            out.at[c, scid, pl.ds(sid * RPS, RPS)])
            plsc.subcore_barrier()

    return sc_agg


def _dot16(a, w):
    return lax.dot_general(a.astype(jnp.bfloat16), w.astype(jnp.bfloat16),
                           (((1,), (0,)), ((), ())),
                           preferred_element_type=jnp.float32)


def _mm_stats_body(agg_ref, deg_ref, base_ref, wl_ref, b_ref,
                   hpre_ref, ssum_ref, ssq_ref, *, n_chunks):
    i = pl.program_id(0)
    cnt16 = deg_ref[0] + deg_ref[1]
    inv = 1.0 / jnp.maximum(cnt16[:, :1], 1.0)
    h = base_ref[...] + b_ref[...]
    for c in range(n_chunks):
        mc = (agg_ref[c, 0] + agg_ref[c, 1]) * inv
        h = h + _dot16(mc, wl_ref[pl.ds(c * 128, 128), :])
    hpre_ref[...] = h

    rows = lax.broadcasted_iota(jnp.int32, (BR, 1), 0) + i * BR
    hm = jnp.where(rows < N_VALID, h, 0.0)

    @pl.when(i == 0)
    def _():
        ssum_ref[...] = jnp.zeros_like(ssum_ref)
        ssq_ref[...] = jnp.zeros_like(ssq_ref)
    ssum_ref[...] += jnp.sum(hm, axis=0, keepdims=True)
    ssq_ref[...] += jnp.sum(hm * hm, axis=0, keepdims=True)


def _matmul_body(x_ref, w_ref, o_ref):
    o_ref[...] = _dot16(x_ref[...], w_ref[...])


def _matmul(x, w):
    din, dout = w.shape
    return pl.pallas_call(
        _matmul_body,
        grid=(GRID,),
        in_specs=[
            pl.BlockSpec((BR, din), lambda i: (i, 0)),
            pl.BlockSpec((din, dout), lambda i: (0, 0)),
        ],
        out_specs=pl.BlockSpec((BR, dout), lambda i: (i, 0)),
        out_shape=jax.ShapeDtypeStruct((NP, dout), jnp.float32),
    )(x, w)


def _mm_stats(agg, deg, base, wl, b, n_chunks, dout):
    return pl.pallas_call(
        functools.partial(_mm_stats_body, n_chunks=n_chunks),
        grid=(GRID,),
        in_specs=[
            pl.BlockSpec((n_chunks, 2, BR, 128), lambda i: (0, 0, i, 0)),
            pl.BlockSpec((2, BR, 128), lambda i: (0, i, 0)),
            pl.BlockSpec((BR, dout), lambda i: (i, 0)),
            pl.BlockSpec((n_chunks * 128, dout), lambda i: (0, 0)),
            pl.BlockSpec((1, dout), lambda i: (0, 0)),
        ],
        out_specs=[
            pl.BlockSpec((BR, dout), lambda i: (i, 0)),
            pl.BlockSpec((1, dout), lambda i: (0, 0)),
            pl.BlockSpec((1, dout), lambda i: (0, 0)),
        ],
        out_shape=[
            jax.ShapeDtypeStruct((NP, dout), jnp.float32),
            jax.ShapeDtypeStruct((1, dout), jnp.float32),
            jax.ShapeDtypeStruct((1, dout), jnp.float32),
        ],
    )(agg, deg, base, wl, b)


def _bn_relu_body(hpre_ref, ssum_ref, ssq_ref, g_ref, bt_ref, *rest, nfuse):
    w_refs = rest[:nfuse]
    h_ref = rest[nfuse]
    t_refs = rest[nfuse + 1:]
    mu = ssum_ref[...] / N_VALID
    var = ssq_ref[...] / N_VALID - mu * mu
    scale = g_ref[...] * lax.rsqrt(var + 1e-5)
    h = jnp.maximum((hpre_ref[...] - mu) * scale + bt_ref[...], 0.0)
    h_ref[...] = h
    for w_ref, t_ref in zip(w_refs, t_refs):
        t_ref[...] = _dot16(h, w_ref[...])


def _bn_relu(hpre, ssum, ssq, gamma, beta, wl_next=None):
    d = hpre.shape[1]
    in_specs = [
        pl.BlockSpec((BR, d), lambda i: (i, 0)),
        pl.BlockSpec((1, d), lambda i: (0, 0)),
        pl.BlockSpec((1, d), lambda i: (0, 0)),
        pl.BlockSpec((1, d), lambda i: (0, 0)),
        pl.BlockSpec((1, d), lambda i: (0, 0)),
    ]
    out_specs = [pl.BlockSpec((BR, d), lambda i: (i, 0))]
    out_shape = [jax.ShapeDtypeStruct((NP, d), jnp.float32)]
    args = [hpre, ssum, ssq, gamma, beta]
    wl_next = wl_next or []
    for w in wl_next:
        dn = w.shape[1]
        in_specs.append(pl.BlockSpec((d, dn), lambda i: (0, 0)))
        out_specs.append(pl.BlockSpec((BR, dn), lambda i: (i, 0)))
        out_shape.append(jax.ShapeDtypeStruct((NP, dn), jnp.float32))
        args.append(w)
    return pl.pallas_call(
        functools.partial(_bn_relu_body, nfuse=len(wl_next)),
        grid=(GRID,),
        in_specs=in_specs, out_specs=out_specs, out_shape=out_shape,
    )(*args)


def _final_body2(agg_ref, deg_ref, base_ref, b_ref, out_ref):
    cnt16 = deg_ref[0] + deg_ref[1]
    inv = 1.0 / jnp.maximum(cnt16[:, :1], 1.0)
    mean = jnp.concatenate(
        [(agg_ref[c, 0] + agg_ref[c, 1]) * inv for c in range(2)], axis=1)
    y = mean + base_ref[...] + b_ref[...]
    m = jnp.max(y, axis=1, keepdims=True)
    e = jnp.exp(y - m)
    s = jnp.sum(e, axis=1, keepdims=True)
    out_ref[...] = y - m - jnp.log(s)


def _final(agg, deg, base, b):
    dout = base.shape[1]
    return pl.pallas_call(
        _final_body2,
        grid=(GRID,),
        in_specs=[
            pl.BlockSpec((2, 2, BR, 128), lambda i: (0, 0, i, 0)),
            pl.BlockSpec((2, BR, 128), lambda i: (0, i, 0)),
            pl.BlockSpec((BR, dout), lambda i: (i, 0)),
            pl.BlockSpec((1, dout), lambda i: (0, 0)),
        ],
        out_specs=pl.BlockSpec((BR, dout), lambda i: (i, 0)),
        out_shape=jax.ShapeDtypeStruct((NP, dout), jnp.float32),
    )(agg, deg, base, b)


_sc_agg_l1 = _make_sc_agg(2, with_deg=True)
_sc_agg_l2 = _make_sc_agg(4, with_deg=False)
_sc_agg_l3 = _make_sc_agg(2, with_deg=False)


def kernel(x, edge_index, W_l1, b_l1, W_r1, gamma1, beta1,
           W_l2, b_l2, W_r2, gamma2, beta2, W_l3, b_l3, W_r3):
    n, din = x.shape
    e = edge_index.shape[1]
    src = edge_index[0].astype(jnp.int32)
    dst = edge_index[1].astype(jnp.int32)
    pad = jnp.full((EP - e,), n, jnp.int32)
    src2d = jnp.concatenate([src, pad]).reshape(NBLK, 128)
    dst2d = jnp.concatenate([dst, pad]).reshape(NBLK, 128)

    xp = jnp.pad(x, ((0, NP - n), (0, 0)))
    t0, t1 = xp[:, :128], xp[:, 128:256]

    z128 = jnp.zeros((128, 128), jnp.float32)
    agg1, deg = _sc_agg_l1(t0, t1, src2d, dst2d, z128)
    r1 = _matmul(xp, W_r1)
    h1pre, s1, q1 = _mm_stats(agg1, deg, r1, W_l1, b_l1.reshape(1, -1), 2, 512)
    h1, r2 = _bn_relu(h1pre, s1, q1, gamma1.reshape(1, -1),
                      beta1.reshape(1, -1), wl_next=[W_r2])

    c1 = [lax.slice(h1, (0, k * 128), (NP, (k + 1) * 128)) for k in range(4)]
    (agg2,) = _sc_agg_l2(*c1, src2d, dst2d, z128)
    h2pre, s2, q2 = _mm_stats(agg2, deg, r2, W_l2, b_l2.reshape(1, -1), 4, 512)
    h2, t3, r3 = _bn_relu(h2pre, s2, q2, gamma2.reshape(1, -1),
                          beta2.reshape(1, -1), wl_next=[W_l3, W_r3])

    c3 = [lax.slice(t3, (0, k * 128), (NP, (k + 1) * 128)) for k in range(2)]
    (agg3,) = _sc_agg_l3(*c3, src2d, dst2d, z128)
    out = _final(agg3, deg, r3, b_l3.reshape(1, -1))
    return out[:n]


# speedup vs baseline: 1.0094x; 1.0094x over previous
import functools

import jax
import jax.numpy as jnp
from jax import lax
from jax.experimental import pallas as pl
from jax.experimental.pallas import tpu as pltpu
from jax.experimental.pallas import tpu_sc as plsc

N_VALID = 10000
NP = 10240
EP = 163840
NBLK = EP // 128
BPT = NBLK // 32
RPS = NP // 16
BR = 512
GRID = NP // BR
NBUF = 2


def _make_sc_agg(n_chunks, with_deg):
    mesh = plsc.VectorSubcoreMesh(core_axis_name="c", subcore_axis_name="s")
    out_type = [jax.ShapeDtypeStruct((n_chunks, 2, NP, 128), jnp.float32)]
    if with_deg:
        out_type.append(jax.ShapeDtypeStruct((2, NP, 128), jnp.float32))
    scratch = [
        pltpu.VMEM((BPT, 128), jnp.int32),
        pltpu.VMEM((BPT, 128), jnp.int32),
        pltpu.VMEM((NBUF, 128, 128), jnp.float32),
        pltpu.VMEM_SHARED((NP, 128), jnp.float32),
    ] + [pltpu.SemaphoreType.DMA] * NBUF

    @functools.partial(pl.kernel, mesh=mesh, out_type=tuple(out_type),
                       scratch_types=tuple(scratch))
    def sc_agg(*refs):
        tables = refs[:n_chunks]
        src_hbm, dst_hbm, z_hbm = (refs[n_chunks], refs[n_chunks + 1],
                                   refs[n_chunks + 2])
        k = n_chunks + 3
        out = refs[k]
        k += 1
        if with_deg:
            deg_out = refs[k]
            k += 1
        src_v, dst_v, rows = refs[k], refs[k + 1], refs[k + 2]
        acc = refs[k + 3]
        sems = refs[k + 4:k + 4 + NBUF]

        scid = lax.axis_index("c")
        sid = lax.axis_index("s")
        tid = scid * 16 + sid

        pltpu.sync_copy(src_hbm.at[pl.ds(tid * BPT, BPT)], src_v)
        pltpu.sync_copy(dst_hbm.at[pl.ds(tid * BPT, BPT)], dst_v)

        def reset_acc():
            for r in range(RPS // 128):
                pltpu.sync_copy(z_hbm, acc.at[pl.ds(sid * RPS + r * 128, 128)])

        if with_deg:
            def orow(i, _):
                for j in range(8):
                    rows[0, i, pl.ds(j * 16, 16)] = jnp.ones((16,), jnp.float32)
                return 0
            lax.fori_loop(0, 128, orow, 0)
            reset_acc()
            plsc.subcore_barrier()

            def deg_body(i, _):
                pltpu.sync_copy(rows.at[0], acc.at[dst_v.at[i]], add=True)
                return 0
            lax.fori_loop(0, BPT, deg_body, 0)
            plsc.subcore_barrier()
            pltpu.sync_copy(acc.at[pl.ds(sid * RPS, RPS)],
                            deg_out.at[scid, pl.ds(sid * RPS, RPS)])
            plsc.subcore_barrier()

        for c in range(n_chunks):
            tab = tables[c]
            for b in range(NBUF):
                pltpu.async_copy(tab.at[src_v.at[b]], rows.at[b], sems[b])
            reset_acc()
            plsc.subcore_barrier()

            def body(j, _, _t=tab):
                for b in range(NBUF):
                    i = j * NBUF + b
                    pltpu.make_async_copy(
                        _t.at[src_v.at[i]], rows.at[b], sems[b]).wait()
                    pltpu.sync_copy(rows.at[b], acc.at[dst_v.at[i]], add=True)

                    @pl.when(i + NBUF < BPT)
                    def _(b=b, i=i):
                        pltpu.async_copy(
                            _t.at[src_v.at[i + NBUF]], rows.at[b], sems[b])
                return 0
            lax.fori_loop(0, BPT // NBUF, body, 0)
            plsc.subcore_barrier()
            pltpu.sync_copy(acc.at[pl.ds(sid * RPS, RPS)],
                            out.at[c, scid, pl.ds(sid * RPS, RPS)])
            plsc.subcore_barrier()

    return sc_agg


def _dot16(a, w):
    return lax.dot_general(a.astype(jnp.bfloat16), w.astype(jnp.bfloat16),
                           (((1,), (0,)), ((), ())),
                           preferred_element_type=jnp.float32)


def _mm_stats_body(agg_ref, deg_ref, base_ref, wl_ref, b_ref,
                   hpre_ref, ssum_ref, ssq_ref, *, n_chunks):
    i = pl.program_id(0)
    cnt16 = deg_ref[0] + deg_ref[1]
    inv = 1.0 / jnp.maximum(cnt16[:, :1], 1.0)
    h = base_ref[...] + b_ref[...]
    for c in range(n_chunks):
        mc = (agg_ref[c, 0] + agg_ref[c, 1]) * inv
        h = h + _dot16(mc, wl_ref[pl.ds(c * 128, 128), :])
    hpre_ref[...] = h

    rows = lax.broadcasted_iota(jnp.int32, (BR, 1), 0) + i * BR
    hm = jnp.where(rows < N_VALID, h, 0.0)

    @pl.when(i == 0)
    def _():
        ssum_ref[...] = jnp.zeros_like(ssum_ref)
        ssq_ref[...] = jnp.zeros_like(ssq_ref)
    ssum_ref[...] += jnp.sum(hm, axis=0, keepdims=True)
    ssq_ref[...] += jnp.sum(hm * hm, axis=0, keepdims=True)


def _matmul_body(x_ref, w_ref, o_ref):
    o_ref[...] = _dot16(x_ref[...], w_ref[...])


def _matmul(x, w):
    din, dout = w.shape
    return pl.pallas_call(
        _matmul_body,
        grid=(GRID,),
        in_specs=[
            pl.BlockSpec((BR, din), lambda i: (i, 0)),
            pl.BlockSpec((din, dout), lambda i: (0, 0)),
        ],
        out_specs=pl.BlockSpec((BR, dout), lambda i: (i, 0)),
        out_shape=jax.ShapeDtypeStruct((NP, dout), jnp.float32),
    )(x, w)


def _mm_stats(agg, deg, base, wl, b, n_chunks, dout):
    return pl.pallas_call(
        functools.partial(_mm_stats_body, n_chunks=n_chunks),
        grid=(GRID,),
        in_specs=[
            pl.BlockSpec((n_chunks, 2, BR, 128), lambda i: (0, 0, i, 0)),
            pl.BlockSpec((2, BR, 128), lambda i: (0, i, 0)),
            pl.BlockSpec((BR, dout), lambda i: (i, 0)),
            pl.BlockSpec((n_chunks * 128, dout), lambda i: (0, 0)),
            pl.BlockSpec((1, dout), lambda i: (0, 0)),
        ],
        out_specs=[
            pl.BlockSpec((BR, dout), lambda i: (i, 0)),
            pl.BlockSpec((1, dout), lambda i: (0, 0)),
            pl.BlockSpec((1, dout), lambda i: (0, 0)),
        ],
        out_shape=[
            jax.ShapeDtypeStruct((NP, dout), jnp.float32),
            jax.ShapeDtypeStruct((1, dout), jnp.float32),
            jax.ShapeDtypeStruct((1, dout), jnp.float32),
        ],
    )(agg, deg, base, wl, b)


def _bn_relu_body(hpre_ref, ssum_ref, ssq_ref, g_ref, bt_ref, *rest, nfuse):
    w_refs = rest[:nfuse]
    h_ref = rest[nfuse]
    t_refs = rest[nfuse + 1:]
    mu = ssum_ref[...] / N_VALID
    var = ssq_ref[...] / N_VALID - mu * mu
    scale = g_ref[...] * lax.rsqrt(var + 1e-5)
    h = jnp.maximum((hpre_ref[...] - mu) * scale + bt_ref[...], 0.0)
    h_ref[...] = h
    for w_ref, t_ref in zip(w_refs, t_refs):
        t_ref[...] = _dot16(h, w_ref[...])


def _bn_relu(hpre, ssum, ssq, gamma, beta, wl_next=None):
    d = hpre.shape[1]
    in_specs = [
        pl.BlockSpec((BR, d), lambda i: (i, 0)),
        pl.BlockSpec((1, d), lambda i: (0, 0)),
        pl.BlockSpec((1, d), lambda i: (0, 0)),
        pl.BlockSpec((1, d), lambda i: (0, 0)),
        pl.BlockSpec((1, d), lambda i: (0, 0)),
    ]
    out_specs = [pl.BlockSpec((BR, d), lambda i: (i, 0))]
    out_shape = [jax.ShapeDtypeStruct((NP, d), jnp.float32)]
    args = [hpre, ssum, ssq, gamma, beta]
    wl_next = wl_next or []
    for w in wl_next:
        dn = w.shape[1]
        in_specs.append(pl.BlockSpec((d, dn), lambda i: (0, 0)))
        out_specs.append(pl.BlockSpec((BR, dn), lambda i: (i, 0)))
        out_shape.append(jax.ShapeDtypeStruct((NP, dn), jnp.float32))
        args.append(w)
    return pl.pallas_call(
        functools.partial(_bn_relu_body, nfuse=len(wl_next)),
        grid=(GRID,),
        in_specs=in_specs, out_specs=out_specs, out_shape=out_shape,
    )(*args)


def _final_body2(agg_ref, deg_ref, base_ref, b_ref, out_ref):
    cnt16 = deg_ref[0] + deg_ref[1]
    inv = 1.0 / jnp.maximum(cnt16[:, :1], 1.0)
    mean = jnp.concatenate(
        [(agg_ref[c, 0] + agg_ref[c, 1]) * inv for c in range(2)], axis=1)
    y = mean + base_ref[...] + b_ref[...]
    m = jnp.max(y, axis=1, keepdims=True)
    e = jnp.exp(y - m)
    s = jnp.sum(e, axis=1, keepdims=True)
    out_ref[...] = y - m - jnp.log(s)


def _final(agg, deg, base, b):
    dout = base.shape[1]
    return pl.pallas_call(
        _final_body2,
        grid=(GRID,),
        in_specs=[
            pl.BlockSpec((2, 2, BR, 128), lambda i: (0, 0, i, 0)),
            pl.BlockSpec((2, BR, 128), lambda i: (0, i, 0)),
            pl.BlockSpec((BR, dout), lambda i: (i, 0)),
            pl.BlockSpec((1, dout), lambda i: (0, 0)),
        ],
        out_specs=pl.BlockSpec((BR, dout), lambda i: (i, 0)),
        out_shape=jax.ShapeDtypeStruct((NP, dout), jnp.float32),
    )(agg, deg, base, b)


_sc_agg_l1 = _make_sc_agg(2, with_deg=True)
_sc_agg_l2 = _make_sc_agg(4, with_deg=False)
_sc_agg_l3 = _make_sc_agg(2, with_deg=False)


def kernel(x, edge_index, W_l1, b_l1, W_r1, gamma1, beta1,
           W_l2, b_l2, W_r2, gamma2, beta2, W_l3, b_l3, W_r3):
    n, din = x.shape
    e = edge_index.shape[1]
    src = edge_index[0].astype(jnp.int32)
    dst = edge_index[1].astype(jnp.int32)
    pad = jnp.full((EP - e,), n, jnp.int32)
    src2d = jnp.concatenate([src, pad]).reshape(NBLK, 128)
    dst2d = jnp.concatenate([dst, pad]).reshape(NBLK, 128)

    xp = jnp.pad(x, ((0, NP - n), (0, 0)))
    t0, t1 = xp[:, :128], xp[:, 128:256]

    z128 = jnp.zeros((128, 128), jnp.float32)
    agg1, deg = _sc_agg_l1(t0, t1, src2d, dst2d, z128)
    r1 = _matmul(xp, W_r1)
    h1pre, s1, q1 = _mm_stats(agg1, deg, r1, W_l1, b_l1.reshape(1, -1), 2, 512)
    h1, r2 = _bn_relu(h1pre, s1, q1, gamma1.reshape(1, -1),
                      beta1.reshape(1, -1), wl_next=[W_r2])

    c1 = [lax.slice(h1, (0, k * 128), (NP, (k + 1) * 128)) for k in range(4)]
    (agg2,) = _sc_agg_l2(*c1, src2d, dst2d, z128)
    h2pre, s2, q2 = _mm_stats(agg2, deg, r2, W_l2, b_l2.reshape(1, -1), 4, 512)
    h2, t3, r3 = _bn_relu(h2pre, s2, q2, gamma2.reshape(1, -1),
                          beta2.reshape(1, -1), wl_next=[W_l3, W_r3])

    c3 = [lax.slice(t3, (0, k * 128), (NP, (k + 1) * 128)) for k in range(2)]
    (agg3,) = _sc_agg_l3(*c3, src2d, dst2d, z128)
    out = _final(agg3, deg, r3, b_l3.reshape(1, -1))
    return out[:n]
